# baseline (speedup 1.0000x reference)
import jax
import jax.numpy as jnp
import numpy as np
from jax import lax
from jax.experimental import pallas as pl
from jax.experimental.pallas import tpu as pltpu

B, S, D = 1, 2048, 1024
H = 16
DH = D // H
P = 64
K = 2
R = 16

S_BLK = 2048
H_BLK = 2
LOG2E = float(np.log2(np.e))


def _hilo(a, axis):
    a_hi = a.astype(jnp.bfloat16)
    a_lo = (a - a_hi.astype(jnp.float32)).astype(jnp.bfloat16)
    return jnp.concatenate([a_hi, a_hi, a_lo], axis=axis)


def _hilo_w(a, axis):
    a_hi = a.astype(jnp.bfloat16)
    a_lo = (a - a_hi.astype(jnp.float32)).astype(jnp.bfloat16)
    return jnp.concatenate([a_hi, a_lo, a_hi], axis=axis)


def _one_head(xh, wq, wr, sk, sv, g):
    qT = lax.dot_general(wq, xh, (((0,), (1,)), ((), ())),
                         preferred_element_type=jnp.float32)
    qT = qT * (LOG2E / (DH ** 0.5))

    rT = lax.dot_general(wr, xh, (((0,), (1,)), ((), ())),
                         preferred_element_type=jnp.float32)

    qTp = _hilo(qT, 0)
    skp = _hilo_w(sk, 1)

    sT = lax.dot_general(skp, qTp, (((1,), (0,)), ((), ())),
                         preferred_element_type=jnp.float32)

    e3 = jnp.exp2(sT)
    e3b = e3.astype(jnp.bfloat16)

    den = jnp.sum(e3.reshape(P, R, S_BLK), axis=1)

    rowid = lax.broadcasted_iota(jnp.int32, (P, S_BLK), 0)
    m1 = jnp.max(rT, axis=0, keepdims=True)
    i1 = jnp.min(jnp.where(rT == m1, rowid, P), axis=0, keepdims=True)
    mask1 = rowid == i1
    rT2 = jnp.where(mask1, -jnp.inf, rT)
    m2 = jnp.max(rT2, axis=0, keepdims=True)
    i2 = jnp.min(jnp.where(rT2 == m2, rowid, P), axis=0, keepdims=True)
    mask2 = rowid == i2
    eg = jnp.exp(m2 - m1)
    g1 = 1.0 / (1.0 + eg)
    g2 = eg * g1
    gateT = jnp.where(mask1, g1, 0.0) + jnp.where(mask2, g2, 0.0)

    gdb = (gateT / den).astype(jnp.bfloat16)
    fullT = (e3b.reshape(P, R, S_BLK) * gdb.reshape(P, 1, S_BLK)
             ).reshape(P * R, S_BLK)

    ohT = lax.dot_general(sv.astype(jnp.bfloat16), fullT,
                          (((0,), (0,)), ((), ())),
                          preferred_element_type=jnp.float32)
    return ohT.astype(jnp.bfloat16)


def _fused_kernel(x_ref, wq_ref, wr_ref, sk_ref, sv_ref, g_ref, wo_ref,
                  b_ref, out_ref, conc_ref):
    j = pl.program_id(1)
    g = g_ref[...]

    for t in range(H_BLK):
        xh = x_ref[:, t * DH:(t + 1) * DH]
        ohT = _one_head(xh, wq_ref[t], wr_ref[t], sk_ref[t], sv_ref[t], g)
        conc_ref[pl.ds((j * H_BLK + t) * DH, DH), :] = ohT

    @pl.when(j == H // H_BLK - 1)
    def _():
        out_ref[...] = lax.dot_general(
            conc_ref[...], wo_ref[...].astype(jnp.bfloat16),
            (((0,), (1,)), ((), ())),
            preferred_element_type=jnp.float32) + b_ref[...]


@jax.jit
def kernel(x, Wq, Wr, state_k, state_v, Wout, b_out):
    x2 = x.reshape(S, D)
    sk = state_k.reshape(H, P * R, DH)
    sv = state_v.reshape(H, P * R, DH)
    grp = jnp.repeat(jnp.eye(P, dtype=jnp.bfloat16), R, axis=0)
    b2 = b_out.reshape(1, D)

    grid = (S // S_BLK, H // H_BLK)

    out = pl.pallas_call(
        _fused_kernel,
        grid=grid,
        in_specs=[
            pl.BlockSpec((S_BLK, H_BLK * DH), lambda i, j: (i, j)),
            pl.BlockSpec((H_BLK, DH, DH), lambda i, j: (j, 0, 0)),
            pl.BlockSpec((H_BLK, DH, P), lambda i, j: (j, 0, 0)),
            pl.BlockSpec((H_BLK, P * R, DH), lambda i, j: (j, 0, 0)),
            pl.BlockSpec((H_BLK, P * R, DH), lambda i, j: (j, 0, 0)),
            pl.BlockSpec((P * R, P), lambda i, j: (0, 0)),
            pl.BlockSpec((D, D), lambda i, j: (0, 0)),
            pl.BlockSpec((1, D), lambda i, j: (0, 0)),
        ],
        out_specs=pl.BlockSpec((S_BLK, D), lambda i, j: (i, 0)),
        out_shape=jax.ShapeDtypeStruct((S, D), jnp.float32),
        scratch_shapes=[pltpu.VMEM((D, S_BLK), jnp.bfloat16)],
        compiler_params=pltpu.CompilerParams(
            dimension_semantics=("parallel", "arbitrary"),
        ),
    )(x2, Wq, Wr, sk, sv, grp, Wout, b2)

    return out.reshape(B, S, D)


# speedup vs baseline: 1.0571x; 1.0571x over previous
import jax
import jax.numpy as jnp
import numpy as np
from jax import lax
from jax.experimental import pallas as pl
from jax.experimental.pallas import tpu as pltpu

B, S, D = 1, 2048, 1024
H = 16
DH = D // H
P = 64
K = 2
R = 16

S_BLK = 2048
H_BLK = 2
LOG2E = float(np.log2(np.e))


def _hilo(a, axis):
    a_hi = a.astype(jnp.bfloat16)
    a_lo = (a - a_hi.astype(jnp.float32)).astype(jnp.bfloat16)
    return jnp.concatenate([a_hi, a_hi, a_lo], axis=axis)


def _hilo_w(a, axis):
    a_hi = a.astype(jnp.bfloat16)
    a_lo = (a - a_hi.astype(jnp.float32)).astype(jnp.bfloat16)
    return jnp.concatenate([a_hi, a_lo, a_hi], axis=axis)


def _one_head(xh, wq, wr, sk, sv, g):
    qT = lax.dot_general(wq, xh, (((0,), (1,)), ((), ())),
                         preferred_element_type=jnp.float32)
    qT = qT * (LOG2E / (DH ** 0.5))

    rT = lax.dot_general(wr, xh, (((0,), (1,)), ((), ())),
                         preferred_element_type=jnp.float32)

    qTp = _hilo(qT, 0)
    skp = _hilo_w(sk, 1)

    sT = lax.dot_general(skp, qTp, (((1,), (0,)), ((), ())),
                         preferred_element_type=jnp.float32)

    e3 = jnp.exp2(sT)
    e3b = e3.astype(jnp.bfloat16)

    den = lax.dot_general(g, e3b, (((0,), (0,)), ((), ())),
                          preferred_element_type=jnp.float32)

    rowid = lax.broadcasted_iota(jnp.int32, (P, S_BLK), 0)
    m1 = jnp.max(rT, axis=0, keepdims=True)
    i1 = jnp.min(jnp.where(rT == m1, rowid, P), axis=0, keepdims=True)
    mask1 = rowid == i1
    rT2 = jnp.where(mask1, -jnp.inf, rT)
    m2 = jnp.max(rT2, axis=0, keepdims=True)
    i2 = jnp.min(jnp.where(rT2 == m2, rowid, P), axis=0, keepdims=True)
    mask2 = rowid == i2
    eg = jnp.exp(m2 - m1)
    g1 = 1.0 / (1.0 + eg)
    g2 = eg * g1
    gateT = jnp.where(mask1, g1, 0.0) + jnp.where(mask2, g2, 0.0)

    gdb = (gateT / den).astype(jnp.bfloat16)
    fullT = (e3b.reshape(P, R, S_BLK) * gdb.reshape(P, 1, S_BLK)
             ).reshape(P * R, S_BLK)

    ohT = lax.dot_general(sv.astype(jnp.bfloat16), fullT,
                          (((0,), (0,)), ((), ())),
                          preferred_element_type=jnp.float32)
    return ohT.astype(jnp.bfloat16)


def _fused_kernel(x_ref, wq_ref, wr_ref, sk_ref, sv_ref, g_ref, wo_ref,
                  b_ref, out_ref, conc_ref):
    j = pl.program_id(1)
    g = g_ref[...]

    for t in range(H_BLK):
        xh = x_ref[:, t * DH:(t + 1) * DH]
        ohT = _one_head(xh, wq_ref[t], wr_ref[t], sk_ref[t], sv_ref[t], g)
        conc_ref[pl.ds((j * H_BLK + t) * DH, DH), :] = ohT

    @pl.when(j == H // H_BLK - 1)
    def _():
        out_ref[...] = lax.dot_general(
            conc_ref[...], wo_ref[...].astype(jnp.bfloat16),
            (((0,), (1,)), ((), ())),
            preferred_element_type=jnp.float32) + b_ref[...]


@jax.jit
def kernel(x, Wq, Wr, state_k, state_v, Wout, b_out):
    x2 = x.reshape(S, D)
    sk = state_k.reshape(H, P * R, DH)
    sv = state_v.reshape(H, P * R, DH)
    grp = jnp.repeat(jnp.eye(P, dtype=jnp.bfloat16), R, axis=0)
    b2 = b_out.reshape(1, D)

    grid = (S // S_BLK, H // H_BLK)

    out = pl.pallas_call(
        _fused_kernel,
        grid=grid,
        in_specs=[
            pl.BlockSpec((S_BLK, H_BLK * DH), lambda i, j: (i, j)),
            pl.BlockSpec((H_BLK, DH, DH), lambda i, j: (j, 0, 0)),
            pl.BlockSpec((H_BLK, DH, P), lambda i, j: (j, 0, 0)),
            pl.BlockSpec((H_BLK, P * R, DH), lambda i, j: (j, 0, 0)),
            pl.BlockSpec((H_BLK, P * R, DH), lambda i, j: (j, 0, 0)),
            pl.BlockSpec((P * R, P), lambda i, j: (0, 0)),
            pl.BlockSpec((D, D), lambda i, j: (0, 0)),
            pl.BlockSpec((1, D), lambda i, j: (0, 0)),
        ],
        out_specs=pl.BlockSpec((S_BLK, D), lambda i, j: (i, 0)),
        out_shape=jax.ShapeDtypeStruct((S, D), jnp.float32),
        scratch_shapes=[pltpu.VMEM((D, S_BLK), jnp.bfloat16)],
        compiler_params=pltpu.CompilerParams(
            dimension_semantics=("parallel", "arbitrary"),
        ),
    )(x2, Wq, Wr, sk, sv, grp, Wout, b2)

    return out.reshape(B, S, D)
